# baseline (speedup 1.0000x reference)
import functools

import jax
import jax.numpy as jnp
from jax import lax
from jax.experimental import pallas as pl
from jax.experimental.pallas import tpu as pltpu
from jax.experimental.pallas import tpu_sc as plsc

_NC = 2
_NS = 16
_NW = _NC * _NS
_L = 16
_BB = 256
_NA = 2


@functools.lru_cache(maxsize=None)
def _make_sc_analyze(N, V):
    bpw = N // _NW
    mesh = plsc.VectorSubcoreMesh(core_axis_name="c", subcore_axis_name="s")

    @functools.partial(
        pl.kernel,
        mesh=mesh,
        out_type=jax.ShapeDtypeStruct((_NW * _L,), jnp.int32),
        scratch_types=[
            pltpu.VMEM((bpw,), jnp.int32),
            pltpu.VMEM((_NW * _L,), jnp.int32),
        ],
        compiler_params=pltpu.CompilerParams(needs_layout_passes=False),
    )
    def body(x_hbm, flags_hbm, idx_v, flag_v):
        cid = lax.axis_index("c")
        sid = lax.axis_index("s")
        wid = sid * _NC + cid
        base = pl.multiple_of(wid * bpw, 8)

        pltpu.sync_copy(x_hbm.at[pl.ds(base, bpw)], idx_v)

        def reduce(j, carry):
            mn, mx = carry
            v = idx_v[pl.ds(j * _L, _L)]
            return (jnp.minimum(mn, jnp.min(v)), jnp.maximum(mx, jnp.max(v)))

        mn, mx = lax.fori_loop(
            0, bpw // _L, reduce,
            (jnp.int32(2 ** 31 - 1), jnp.int32(-(2 ** 31))))

        umn = jnp.minimum(mn, jnp.int32(0))
        umx = jnp.minimum(mx, jnp.int32(0))
        flag = jnp.where(umn == umx, jnp.int32(1), jnp.int32(0))

        lane = lax.broadcasted_iota(jnp.int32, (_L,), 0)
        fbase = pl.multiple_of(wid * _L, 8)
        flag_v[pl.ds(fbase, _L)] = jnp.where(lane < 8, flag, umn)
        pltpu.sync_copy(flag_v.at[pl.ds(fbase, _L)],
                        flags_hbm.at[pl.ds(fbase, _L)])

    return body


def _materialize_block(flag, u, table_ref, xb_ref, out_ref, col_v, cache_s,
                       i, F, V, D):
    dims = (((0,), (0,)), ((), ()))
    dims_b = (((0,), (1,)), ((), ()))

    @pl.when(i == 0)
    def _init():
        cache_s[0] = jnp.int32(0)

    @pl.when(flag == 1)
    def _broadcast():
        stale = jnp.logical_or(cache_s[0] != 1, cache_s[1] != u)

        @pl.when(stale)
        def _compute():
            viota = lax.broadcasted_iota(jnp.int32, (V, _BB), 0)
            oh = (viota == u).astype(jnp.float32)
            col_v[...] = lax.dot_general(
                table_ref[...], oh, dims,
                precision=lax.Precision.HIGHEST,
                preferred_element_type=jnp.float32)
            cache_s[0] = jnp.int32(1)
            cache_s[1] = u

        out_ref[...] = jnp.broadcast_to(col_v[...][None], (F, D, _BB))

    @pl.when(flag != 1)
    def _general():
        biota = lax.broadcasted_iota(jnp.int32, (_BB, V), 1)
        fiota = lax.broadcasted_iota(jnp.int32, (_BB, F), 1)
        xv = jnp.minimum(xb_ref[...], 0)

        def per_field(f, c):
            eff = jnp.sum(jnp.where(fiota == f, xv, 0), axis=1,
                          keepdims=True)
            oh = (biota == eff).astype(jnp.float32)
            out_ref[pl.ds(f, 1)] = lax.dot_general(
                table_ref[...], oh, dims_b,
                precision=lax.Precision.HIGHEST,
                preferred_element_type=jnp.float32)[None]
            return c

        lax.fori_loop(0, F, per_field, 0)


@functools.lru_cache(maxsize=None)
def _make_tc_materialize_a(B, F, V, D):
    def body(table_ref, xa_ref, out_ref, col_v, cache_s):
        i = pl.program_id(0)

        @pl.when(i == 0)
        def _analyze():
            mn = jnp.min(xa_ref[...])
            mx = jnp.max(xa_ref[...])
            u = jnp.minimum(mn, jnp.int32(0))
            cache_s[3] = u
            cache_s[2] = jnp.where(jnp.minimum(mx, jnp.int32(0)) == u,
                                   jnp.int32(1), jnp.int32(0))

        xb = xa_ref.at[pl.ds(i * _BB, _BB), :]
        _materialize_block(cache_s[2], cache_s[3], table_ref, xb, out_ref,
                           col_v, cache_s, i, F, V, D)

    return pl.pallas_call(
        body,
        grid=(_NA,),
        in_specs=[
            pl.BlockSpec((V, D), lambda i: (0, 0)),
            pl.BlockSpec((_NA * _BB, F), lambda i: (0, 0)),
        ],
        out_specs=pl.BlockSpec((F, D, _BB), lambda i: (0, 0, i)),
        out_shape=jax.ShapeDtypeStruct((F, D, B), jnp.float32),
        scratch_shapes=[
            pltpu.VMEM((D, _BB), jnp.float32),
            pltpu.SMEM((4,), jnp.int32),
        ],
        compiler_params=pltpu.CompilerParams(
            dimension_semantics=("arbitrary",)),
    )


@functools.lru_cache(maxsize=None)
def _make_tc_materialize_b(B, F, V, D):
    grid = B // _BB - _NA

    def body(flags_s, table_ref, xb_ref, prev_ref, out_ref, col_v, cache_s):
        i = pl.program_id(0)

        u = flags_s[8]

        def red(w, gf):
            fw = flags_s[w * _L]
            uw = flags_s[w * _L + 8]
            return gf & jnp.where((fw == 1) & (uw == u), 1, 0)

        flag = lax.fori_loop(0, _NW, red, jnp.int32(1))
        _materialize_block(flag, u, table_ref, xb_ref, out_ref, col_v,
                           cache_s, i, F, V, D)

    return pl.pallas_call(
        body,
        grid=(grid,),
        in_specs=[
            pl.BlockSpec(memory_space=pltpu.SMEM),
            pl.BlockSpec((V, D), lambda i: (0, 0)),
            pl.BlockSpec((_BB, F), lambda i: (i + _NA, 0)),
            pl.BlockSpec(memory_space=pl.ANY),
        ],
        out_specs=pl.BlockSpec((F, D, _BB), lambda i: (0, 0, i + _NA)),
        out_shape=jax.ShapeDtypeStruct((F, D, B), jnp.float32),
        scratch_shapes=[
            pltpu.VMEM((D, _BB), jnp.float32),
            pltpu.SMEM((2,), jnp.int32),
        ],
        input_output_aliases={3: 0},
        compiler_params=pltpu.CompilerParams(
            dimension_semantics=("arbitrary",)),
    )


def kernel(x, table):
    B, F = x.shape
    V, D = table.shape
    xf = x.reshape(B * F)
    flags = _make_sc_analyze(B * F, V)(xf)
    out_a = _make_tc_materialize_a(B, F, V, D)(table, x)
    out_fdb = _make_tc_materialize_b(B, F, V, D)(flags, table, x, out_a)
    return out_fdb.transpose(2, 0, 1)


# speedup vs baseline: 1.0305x; 1.0305x over previous
import functools

import jax
import jax.numpy as jnp
from jax import lax
from jax.experimental import pallas as pl
from jax.experimental.pallas import tpu as pltpu
from jax.experimental.pallas import tpu_sc as plsc

_NC = 2
_NS = 16
_NW = _NC * _NS
_L = 16
_BB = 256
_NA = 2


@functools.lru_cache(maxsize=None)
def _make_sc_analyze(N, V):
    bpw = N // _NW
    mesh = plsc.VectorSubcoreMesh(core_axis_name="c", subcore_axis_name="s")

    @functools.partial(
        pl.kernel,
        mesh=mesh,
        out_type=jax.ShapeDtypeStruct((_NW * _L,), jnp.int32),
        scratch_types=[
            pltpu.VMEM((bpw,), jnp.int32),
            pltpu.VMEM((_NW * _L,), jnp.int32),
        ],
        compiler_params=pltpu.CompilerParams(needs_layout_passes=False),
    )
    def body(x_hbm, flags_hbm, idx_v, flag_v):
        cid = lax.axis_index("c")
        sid = lax.axis_index("s")
        wid = sid * _NC + cid
        base = pl.multiple_of(wid * bpw, 8)

        pltpu.sync_copy(x_hbm.at[pl.ds(base, bpw)], idx_v)

        def reduce(j, carry):
            mn, mx = carry
            v = idx_v[pl.ds(j * _L, _L)]
            return (jnp.minimum(mn, jnp.min(v)), jnp.maximum(mx, jnp.max(v)))

        mn, mx = lax.fori_loop(
            0, bpw // _L, reduce,
            (jnp.int32(2 ** 31 - 1), jnp.int32(-(2 ** 31))))

        umn = jnp.minimum(mn, jnp.int32(0))
        umx = jnp.minimum(mx, jnp.int32(0))
        flag = jnp.where(umn == umx, jnp.int32(1), jnp.int32(0))

        lane = lax.broadcasted_iota(jnp.int32, (_L,), 0)
        fbase = pl.multiple_of(wid * _L, 8)
        flag_v[pl.ds(fbase, _L)] = jnp.where(lane < 8, flag, umn)
        pltpu.sync_copy(flag_v.at[pl.ds(fbase, _L)],
                        flags_hbm.at[pl.ds(fbase, _L)])

    return body


def _materialize_block(flag, u, table_ref, xt_ref, out_ref, col_v, cache_s,
                       i, F, V, D):
    dims = (((0,), (0,)), ((), ()))

    @pl.when(i == 0)
    def _init():
        cache_s[0] = jnp.int32(0)

    @pl.when(flag == 1)
    def _broadcast():
        stale = jnp.logical_or(cache_s[0] != 1, cache_s[1] != u)

        @pl.when(stale)
        def _compute():
            viota = lax.broadcasted_iota(jnp.int32, (V, _BB), 0)
            oh = (viota == u).astype(jnp.float32)
            col_v[...] = lax.dot_general(
                table_ref[...], oh, dims,
                precision=lax.Precision.HIGHEST,
                preferred_element_type=jnp.float32)
            cache_s[0] = jnp.int32(1)
            cache_s[1] = u

        out_ref[...] = jnp.broadcast_to(col_v[...][None], (F, D, _BB))

    @pl.when(flag != 1)
    def _general():
        viota = lax.broadcasted_iota(jnp.int32, (V, _BB), 0)

        def per_field(f, c):
            eff = jnp.minimum(xt_ref[pl.ds(f, 1), :], 0)
            oh = (viota == eff).astype(jnp.float32)
            out_ref[pl.ds(f, 1)] = lax.dot_general(
                table_ref[...], oh, dims,
                precision=lax.Precision.HIGHEST,
                preferred_element_type=jnp.float32)[None]
            return c

        lax.fori_loop(0, F, per_field, 0)


@functools.lru_cache(maxsize=None)
def _make_tc_materialize_a(B, F, V, D):
    def body(table_ref, xa_ref, out_ref, col_v, cache_s):
        i = pl.program_id(0)

        @pl.when(i == 0)
        def _analyze():
            mn = jnp.min(xa_ref[...])
            mx = jnp.max(xa_ref[...])
            u = jnp.minimum(mn, jnp.int32(0))
            cache_s[3] = u
            cache_s[2] = jnp.where(jnp.minimum(mx, jnp.int32(0)) == u,
                                   jnp.int32(1), jnp.int32(0))

        xb = xa_ref.at[:, pl.ds(i * _BB, _BB)]
        _materialize_block(cache_s[2], cache_s[3], table_ref, xb, out_ref,
                           col_v, cache_s, i, F, V, D)

    return pl.pallas_call(
        body,
        grid=(_NA,),
        in_specs=[
            pl.BlockSpec((V, D), lambda i: (0, 0)),
            pl.BlockSpec((F, _NA * _BB), lambda i: (0, 0)),
        ],
        out_specs=pl.BlockSpec((F, D, _BB), lambda i: (0, 0, i)),
        out_shape=jax.ShapeDtypeStruct((F, D, B), jnp.float32),
        scratch_shapes=[
            pltpu.VMEM((D, _BB), jnp.float32),
            pltpu.SMEM((4,), jnp.int32),
        ],
        compiler_params=pltpu.CompilerParams(
            dimension_semantics=("arbitrary",)),
    )


@functools.lru_cache(maxsize=None)
def _make_tc_materialize_b(B, F, V, D):
    grid = B // _BB - _NA

    def body(flags_s, table_ref, xt_ref, prev_ref, out_ref, col_v, cache_s):
        i = pl.program_id(0)

        u = flags_s[8]

        def red(w, gf):
            fw = flags_s[w * _L]
            uw = flags_s[w * _L + 8]
            return gf & jnp.where((fw == 1) & (uw == u), 1, 0)

        flag = lax.fori_loop(0, _NW, red, jnp.int32(1))
        _materialize_block(flag, u, table_ref, xt_ref, out_ref, col_v,
                           cache_s, i, F, V, D)

    return pl.pallas_call(
        body,
        grid=(grid,),
        in_specs=[
            pl.BlockSpec(memory_space=pltpu.SMEM),
            pl.BlockSpec((V, D), lambda i: (0, 0)),
            pl.BlockSpec((F, _BB), lambda i: (0, i + _NA)),
            pl.BlockSpec(memory_space=pl.ANY),
        ],
        out_specs=pl.BlockSpec((F, D, _BB), lambda i: (0, 0, i + _NA)),
        out_shape=jax.ShapeDtypeStruct((F, D, B), jnp.float32),
        scratch_shapes=[
            pltpu.VMEM((D, _BB), jnp.float32),
            pltpu.SMEM((2,), jnp.int32),
        ],
        input_output_aliases={3: 0},
        compiler_params=pltpu.CompilerParams(
            dimension_semantics=("arbitrary",)),
    )


def kernel(x, table):
    B, F = x.shape
    V, D = table.shape
    xt = x.T
    xtf = xt.reshape(B * F)
    flags = _make_sc_analyze(B * F, V)(xtf)
    out_a = _make_tc_materialize_a(B, F, V, D)(table, xt)
    out_fdb = _make_tc_materialize_b(B, F, V, D)(flags, table, xt, out_a)
    return out_fdb.transpose(2, 0, 1)


# speedup vs baseline: 1.0345x; 1.0039x over previous
import functools

import jax
import jax.numpy as jnp
from jax import lax
from jax.experimental import pallas as pl
from jax.experimental.pallas import tpu as pltpu
from jax.experimental.pallas import tpu_sc as plsc

_NC = 2
_NS = 16
_NW = _NC * _NS
_L = 16
_BB = 256


@functools.lru_cache(maxsize=None)
def _make_sc_analyze(N, V):
    bpw = N // _NW
    mesh = plsc.VectorSubcoreMesh(core_axis_name="c", subcore_axis_name="s")

    @functools.partial(
        pl.kernel,
        mesh=mesh,
        out_type=jax.ShapeDtypeStruct((_NW * _L,), jnp.int32),
        scratch_types=[
            pltpu.VMEM((bpw,), jnp.int32),
            pltpu.VMEM((_NW * _L,), jnp.int32),
        ],
        compiler_params=pltpu.CompilerParams(needs_layout_passes=False),
    )
    def body(x_hbm, flags_hbm, idx_v, flag_v):
        cid = lax.axis_index("c")
        sid = lax.axis_index("s")
        wid = sid * _NC + cid
        base = pl.multiple_of(wid * bpw, 8)

        pltpu.sync_copy(x_hbm.at[pl.ds(base, bpw)], idx_v)

        def reduce(j, carry):
            mn, mx = carry
            v = idx_v[pl.ds(j * _L, _L)]
            return (jnp.minimum(mn, jnp.min(v)), jnp.maximum(mx, jnp.max(v)))

        mn, mx = lax.fori_loop(
            0, bpw // _L, reduce,
            (jnp.int32(2 ** 31 - 1), jnp.int32(-(2 ** 31))))

        umn = jnp.minimum(mn, jnp.int32(0))
        umx = jnp.minimum(mx, jnp.int32(0))
        flag = jnp.where(umn == umx, jnp.int32(1), jnp.int32(0))

        lane = lax.broadcasted_iota(jnp.int32, (_L,), 0)
        fbase = pl.multiple_of(wid * _L, 8)
        flag_v[pl.ds(fbase, _L)] = jnp.where(lane < 8, flag, umn)
        pltpu.sync_copy(flag_v.at[pl.ds(fbase, _L)],
                        flags_hbm.at[pl.ds(fbase, _L)])

    return body


@functools.lru_cache(maxsize=None)
def _make_tc_materialize(B, F, V, D):
    grid = B // _BB
    dims = (((0,), (0,)), ((), ()))

    def body(flags_s, table_ref, xt_ref, out_ref, col_v, cache_s):
        i = pl.program_id(0)
        viota = lax.broadcasted_iota(jnp.int32, (V, _BB), 0)

        @pl.when(i == 0)
        def _init():
            cache_s[0] = jnp.int32(0)

            u0 = flags_s[8]

            def red(w, gf):
                fw = flags_s[w * _L]
                uw = flags_s[w * _L + 8]
                return gf & jnp.where((fw == 1) & (uw == u0), 1, 0)

            cache_s[2] = lax.fori_loop(0, _NW, red, jnp.int32(1))
            cache_s[3] = u0

        flag = cache_s[2]
        u = cache_s[3]

        @pl.when(flag == 1)
        def _broadcast():
            stale = jnp.logical_or(cache_s[0] != 1, cache_s[1] != u)

            @pl.when(stale)
            def _compute():
                oh = (viota == u).astype(jnp.float32)
                col_v[...] = lax.dot_general(
                    table_ref[...], oh, dims,
                    precision=lax.Precision.HIGHEST,
                    preferred_element_type=jnp.float32)
                cache_s[0] = jnp.int32(1)
                cache_s[1] = u

            out_ref[...] = jnp.broadcast_to(col_v[...][None], (F, D, _BB))

        @pl.when(flag != 1)
        def _general():
            def per_field(f, c):
                eff = jnp.minimum(xt_ref[pl.ds(f, 1), :], 0)
                oh = (viota == eff).astype(jnp.float32)
                out_ref[pl.ds(f, 1)] = lax.dot_general(
                    table_ref[...], oh, dims,
                    precision=lax.Precision.HIGHEST,
                    preferred_element_type=jnp.float32)[None]
                return c

            lax.fori_loop(0, F, per_field, 0)

    grid_spec = pltpu.PrefetchScalarGridSpec(
        num_scalar_prefetch=1,
        grid=(grid,),
        in_specs=[
            pl.BlockSpec((V, D), lambda i, s: (0, 0)),
            pl.BlockSpec((F, _BB), lambda i, s: (0, i)),
        ],
        out_specs=pl.BlockSpec((F, D, _BB), lambda i, s: (0, 0, i)),
        scratch_shapes=[
            pltpu.VMEM((D, _BB), jnp.float32),
            pltpu.SMEM((4,), jnp.int32),
        ],
    )
    return pl.pallas_call(
        body,
        grid_spec=grid_spec,
        out_shape=jax.ShapeDtypeStruct((F, D, B), jnp.float32),
        compiler_params=pltpu.CompilerParams(
            dimension_semantics=("arbitrary",)),
    )


def kernel(x, table):
    B, F = x.shape
    V, D = table.shape
    xt = x.T
    xtf = xt.reshape(B * F)
    flags = _make_sc_analyze(B * F, V)(xtf)
    out_fdb = _make_tc_materialize(B, F, V, D)(flags, table, xt)
    return out_fdb.transpose(2, 0, 1)
